# baseline (device time: 22317 ns/iter reference)
import jax
import jax.numpy as jnp
from jax import lax
from jax.experimental import pallas as pl
from jax.experimental.pallas import tpu as pltpu

N_DEV = 4
N_EXPERTS = 8
E_PER_DEV = N_EXPERTS // N_DEV


def kernel(x, router_W, route_idx, expert_W):
    n_tok, d_model = x.shape
    _, d_ff = expert_W.shape[1:]

    def body(x_ref, rw_ref, idx_ref, ew_ref, out_ref, gather_ref,
             send_sems, recv_sems):
        my = lax.axis_index("i")

        barrier = pltpu.get_barrier_semaphore()
        for k in range(1, N_DEV):
            pl.semaphore_signal(
                barrier, inc=1,
                device_id=((my + k) % N_DEV,),
                device_id_type=pl.DeviceIdType.MESH,
            )
        pl.semaphore_wait(barrier, N_DEV - 1)

        gather_ref[my] = ew_ref[...].astype(jnp.bfloat16)
        sends = []
        for k in range(1, N_DEV):
            rdma = pltpu.make_async_remote_copy(
                src_ref=gather_ref.at[my],
                dst_ref=gather_ref.at[my],
                send_sem=send_sems.at[k - 1],
                recv_sem=recv_sems.at[my],
                device_id=((my + k) % N_DEV,),
                device_id_type=pl.DeviceIdType.MESH,
            )
            rdma.start()
            sends.append(rdma)

        scores = lax.dot_general(
            x_ref[...], rw_ref[...],
            dimension_numbers=(((1,), (0,)), ((), ())),
            precision=lax.Precision.HIGHEST,
            preferred_element_type=jnp.float32,
        )
        e_iota = lax.broadcasted_iota(jnp.int32, (n_tok, N_EXPERTS), 1)
        m0 = (e_iota == idx_ref[:, 0:1]).astype(jnp.float32)
        m1 = (e_iota == idx_ref[:, 1:2]).astype(jnp.float32)
        s0 = jnp.sum(scores * m0, axis=1, keepdims=True)
        s1 = jnp.sum(scores * m1, axis=1, keepdims=True)
        g0 = 1.0 / (1.0 + jnp.exp(s1 - s0))
        w = g0 * m0 + (1.0 - g0) * m1

        xb = x_ref[...].astype(jnp.bfloat16)

        acc = jnp.zeros((n_tok, d_ff), jnp.float32)
        for src in range(N_DEV):
            @pl.when(src != my)
            def _():
                pltpu.make_async_remote_copy(
                    src_ref=gather_ref.at[src],
                    dst_ref=gather_ref.at[src],
                    send_sem=send_sems.at[0],
                    recv_sem=recv_sems.at[src],
                    device_id=(src,),
                    device_id_type=pl.DeviceIdType.MESH,
                ).wait_recv()
            for j in range(E_PER_DEV):
                e = src * E_PER_DEV + j
                y = jnp.dot(xb, gather_ref[src, j],
                            preferred_element_type=jnp.float32)
                acc = acc + w[:, e:e + 1] * y
        out_ref[...] = acc

        for rdma in sends:
            rdma.wait_send()

    return pl.pallas_call(
        body,
        out_shape=jax.ShapeDtypeStruct((n_tok, d_ff), jnp.float32),
        in_specs=[pl.BlockSpec(memory_space=pltpu.VMEM)] * 4,
        out_specs=pl.BlockSpec(memory_space=pltpu.VMEM),
        scratch_shapes=[
            pltpu.VMEM((N_DEV, E_PER_DEV, d_model, d_ff), jnp.bfloat16),
            pltpu.SemaphoreType.DMA((N_DEV - 1,)),
            pltpu.SemaphoreType.DMA((N_DEV,)),
        ],
        compiler_params=pltpu.CompilerParams(collective_id=0),
    )(x, router_W, route_idx, expert_W)
